# baseline (speedup 1.0000x reference)
import functools

import jax
import jax.numpy as jnp
import numpy as np
from jax import lax
from jax.experimental import pallas as pl
from jax.experimental.pallas import tpu as pltpu
from jax.experimental.pallas import tpu_sc as plsc

N = 10000
D = 128
E = 320000
CHUNK = 128
GCH = 64
NWORKERS = 32
ROWS_PER_W = 80
IDXBLK = 40
EP = NWORKERS * ROWS_PER_W * CHUNK
PAD = EP - E
NSUB = 16
NP = 10240
STRIPE = NP // NSUB


def _sc_body(nodes_h, snd_h, rcv_h, zp_h, zd_h, one_h, pout_h, dout_h,
             p_sh, d_sh, snd_v, rcv_v, rows0, rows1, rows2, rows3, ones_v,
             sem0, sem1, sem2, sem3):
    cid = lax.axis_index("c")
    sid = lax.axis_index("s")
    wid = sid * 2 + cid
    rows_b = (rows0, rows1, rows2, rows3)
    sem_b = (sem0, sem1, sem2, sem3)

    pltpu.sync_copy(one_h, ones_v)
    zcopies = [(zd_h, d_sh.at[pl.ds(sid * STRIPE, STRIPE)])]
    for k in range(STRIPE // CHUNK):
        zcopies.append(
            (zp_h, p_sh.at[pl.ds(sid * STRIPE + k * CHUNK, CHUNK)]))
    for i, (src, dst) in enumerate(zcopies):
        pltpu.async_copy(src, dst, sem_b[i % 4])
    for i, (src, dst) in enumerate(zcopies):
        pltpu.make_async_copy(src, dst, sem_b[i % 4]).wait()
    plsc.subcore_barrier()

    def stage_idx(h):
        src = pl.ds(wid * ROWS_PER_W + h * IDXBLK, IDXBLK)
        pltpu.sync_copy(snd_h.at[src], snd_v)
        pltpu.sync_copy(rcv_h.at[src], rcv_v)

    def fire(t, b):
        idx = snd_v.at[t // 2, pl.ds((t % 2) * GCH, GCH)]
        pltpu.async_copy(nodes_h.at[idx], rows_b[b], sem_b[b])

    def drain_scatter(t, b):
        idx = snd_v.at[t // 2, pl.ds((t % 2) * GCH, GCH)]
        pltpu.make_async_copy(nodes_h.at[idx], rows_b[b], sem_b[b]).wait()
        ridx = rcv_v.at[t // 2, pl.ds((t % 2) * GCH, GCH)]
        pltpu.sync_copy(rows_b[b], p_sh.at[ridx], add=True)
        pltpu.sync_copy(ones_v, d_sh.at[ridx], add=True)

    TOPS = 2 * IDXBLK

    def half(h, carry):
        stage_idx(h)
        for k in range(4):
            fire(k, k)

        def group(g, c):
            for k in range(4):
                drain_scatter(4 * g + k, k)
                fire(4 * g + 4 + k, k)
            return c

        lax.fori_loop(0, TOPS // 4 - 2, group, carry)
        for k in range(4):
            drain_scatter(TOPS - 8 + k, k)
            fire(TOPS - 4 + k, k)
        for k in range(4):
            drain_scatter(TOPS - 4 + k, k)
        return carry

    lax.fori_loop(0, ROWS_PER_W // IDXBLK, half, 0)
    plsc.subcore_barrier()

    wcopies = [(d_sh.at[pl.ds(sid * STRIPE, STRIPE)],
                dout_h.at[pl.ds(cid * NP + sid * STRIPE, STRIPE)])]
    for k in range(STRIPE // CHUNK):
        src = pl.ds(sid * STRIPE + k * CHUNK, CHUNK)
        dst = pl.ds(cid * NP + sid * STRIPE + k * CHUNK, CHUNK)
        wcopies.append((p_sh.at[src], pout_h.at[dst]))
    for i, (src, dst) in enumerate(wcopies):
        pltpu.async_copy(src, dst, sem_b[i % 4])
    for i, (src, dst) in enumerate(wcopies):
        pltpu.make_async_copy(src, dst, sem_b[i % 4]).wait()


_sc_scatter = functools.partial(
    pl.kernel,
    mesh=plsc.VectorSubcoreMesh(core_axis_name="c", subcore_axis_name="s"),
    out_type=[
        jax.ShapeDtypeStruct((2 * NP, D), jnp.float32),
        jax.ShapeDtypeStruct((2 * NP,), jnp.float32),
    ],
    scratch_types=[
        pltpu.VMEM_SHARED((NP, D), jnp.float32),
        pltpu.VMEM_SHARED((NP,), jnp.float32),
        pltpu.VMEM((IDXBLK, CHUNK), jnp.int32),
        pltpu.VMEM((IDXBLK, CHUNK), jnp.int32),
        pltpu.VMEM((GCH, D), jnp.float32),
        pltpu.VMEM((GCH, D), jnp.float32),
        pltpu.VMEM((GCH, D), jnp.float32),
        pltpu.VMEM((GCH, D), jnp.float32),
        pltpu.VMEM((GCH,), jnp.float32),
        pltpu.SemaphoreType.DMA,
        pltpu.SemaphoreType.DMA,
        pltpu.SemaphoreType.DMA,
        pltpu.SemaphoreType.DMA,
    ],
)(_sc_body)

_PAD_RCV = np.int32(N) + (np.arange(PAD, dtype=np.int32) % np.int32(NP - N))
_PAD_SND = (np.arange(PAD, dtype=np.int32) * np.int32(1009)) % np.int32(N)


def _tc_body(p_ref, d_ref, nodes_ref, w_ref, b_ref, o_ref):
    u = p_ref[:N, :] + p_ref[NP:NP + N, :]
    deg = (d_ref[:N] + d_ref[NP:NP + N]).reshape(N, 1)
    w1 = w_ref[:D, :]
    w2 = w_ref[D:, :]
    z = jnp.dot(nodes_ref[...], w2, preferred_element_type=jnp.float32)
    out = jnp.dot(u, w1 - w2, preferred_element_type=jnp.float32)
    o_ref[...] = out + deg * (z + b_ref[...])


def kernel(nodes, senders, receivers, W, b):
    snd = jnp.concatenate(
        [senders.astype(jnp.int32), jnp.asarray(_PAD_SND)]
    ).reshape(EP // CHUNK, CHUNK)
    rcv = jnp.concatenate(
        [receivers.astype(jnp.int32), jnp.asarray(_PAD_RCV)]
    ).reshape(EP // CHUNK, CHUNK)
    zp = jnp.zeros((CHUNK, D), jnp.float32)
    zd = jnp.zeros((STRIPE,), jnp.float32)
    one = jnp.ones((GCH,), jnp.float32)
    p01, d01 = _sc_scatter(nodes, snd, rcv, zp, zd, one)
    out = pl.pallas_call(
        _tc_body,
        out_shape=jax.ShapeDtypeStruct((N, D), jnp.float32),
    )(p01, d01, nodes, W, b.reshape(1, D))
    return out
